# baseline (speedup 1.0000x reference)
import functools

import jax
import jax.numpy as jnp
from jax import lax
from jax.experimental import pallas as pl
from jax.experimental.pallas import tpu as pltpu
from jax.experimental.pallas import tpu_sc as plsc

NBINS = 2048
NC = 2
NS = 16
L = 16
NW = NC * NS
N = 16777216
PER_W = N // NW
CHUNK = 32768
NCHUNK = PER_W // CHUNK

_MM_MAJ = N // (8 * 128)
_MM_BLK = 512
_MM_GRID = _MM_MAJ // _MM_BLK


def _mm_body(x_ref, mm_ref, mn_ref, mx_ref, amn_ref, amx_ref):
    i = pl.program_id(0)

    @pl.when(i == 0)
    def _init():
        amn_ref[...] = jnp.full((8, 128), jnp.inf, jnp.float32)
        amx_ref[...] = jnp.full((8, 128), -jnp.inf, jnp.float32)

    v = x_ref[...]
    amn_ref[...] = jnp.minimum(amn_ref[...], jnp.min(v, axis=0))
    amx_ref[...] = jnp.maximum(amx_ref[...], jnp.max(v, axis=0))

    @pl.when(i == _MM_GRID - 1)
    def _fini():
        mn = jnp.min(amn_ref[...])
        mx = jnp.max(amx_ref[...])
        mm_ref[...] = jnp.stack(
            [jnp.full((L,), mn, jnp.float32), jnp.full((L,), mx, jnp.float32)])
        mn_ref[0, 0] = mn
        mx_ref[0, 0] = mx


_minmax = pl.pallas_call(
    _mm_body,
    grid=(_MM_GRID,),
    in_specs=[pl.BlockSpec((_MM_BLK, 8, 128), lambda i: (i, 0, 0))],
    out_specs=(
        pl.BlockSpec((2, L), lambda i: (0, 0)),
        pl.BlockSpec(memory_space=pltpu.SMEM),
        pl.BlockSpec(memory_space=pltpu.SMEM),
    ),
    out_shape=(
        jax.ShapeDtypeStruct((2, L), jnp.float32),
        jax.ShapeDtypeStruct((1, 1), jnp.float32),
        jax.ShapeDtypeStruct((1, 1), jnp.float32),
    ),
    scratch_shapes=[
        pltpu.VMEM((8, 128), jnp.float32),
        pltpu.VMEM((8, 128), jnp.float32),
    ],
)

_mesh = plsc.VectorSubcoreMesh(core_axis_name="c", subcore_axis_name="s")


@functools.partial(
    pl.kernel,
    out_type=jax.ShapeDtypeStruct((NW, NBINS), jnp.float32),
    mesh=_mesh,
    compiler_params=pltpu.CompilerParams(needs_layout_passes=False),
    scratch_types=[
        pltpu.VMEM((2, CHUNK), jnp.float32),
        pltpu.VMEM((L * NBINS,), jnp.float32),
        pltpu.VMEM((NBINS,), jnp.float32),
        pltpu.VMEM((2, L), jnp.float32),
        pltpu.SemaphoreType.DMA,
        pltpu.SemaphoreType.DMA,
    ],
)
def _hist_k(x_hbm, mm_hbm, hists_hbm, buf, hist, hrow, mmv, sem0, sem1):
    wid = lax.axis_index("s") * NC + lax.axis_index("c")
    base = wid * PER_W
    sems = [sem0, sem1]
    cps = [None, None]
    cps[0] = pltpu.async_copy(x_hbm.at[pl.ds(base, CHUNK)], buf.at[0], sem0)

    pltpu.sync_copy(mm_hbm, mmv)
    mn_vec = mmv[0, :]
    mx_vec = mmv[1, :]
    bw = (mx_vec - mn_vec) * jnp.float32(1.0 / NBINS)
    safe_bw = jnp.where(bw <= 0, jnp.float32(1.0), bw)
    inv_vec = jnp.float32(1.0) / safe_bw

    zero = jnp.zeros((L,), jnp.float32)

    @plsc.parallel_loop(0, (L * NBINS) // L, 1, unroll=8)
    def _zero(j):
        hist[pl.ds(j * L, L)] = zero

    lane = lax.iota(jnp.int32, L)
    ones = jnp.ones((L,), jnp.float32)
    for c in range(NCHUNK):
        b = c % 2
        if c + 1 < NCHUNK:
            cps[1 - b] = pltpu.async_copy(
                x_hbm.at[pl.ds(base + (c + 1) * CHUNK, CHUNK)],
                buf.at[1 - b], sems[1 - b])
        cps[b].wait()

        @plsc.parallel_loop(0, CHUNK // L, 1, unroll=16)
        def _scan(i, b=b):
            v = buf[b, pl.ds(i * L, L)]
            q = jnp.minimum((v - mn_vec) * inv_vec, jnp.float32(NBINS - 1))
            plsc.addupdate_scatter(
                hist, [(q.astype(jnp.int32) << 4) | lane], ones)

    perms = [(lane << 4) | ((lane + k) & (L - 1)) for k in range(L)]

    @plsc.parallel_loop(0, NBINS // L, 1, unroll=2)
    def _comb(j):
        bb = j * (L * L)
        acc = plsc.load_gather(hist, [bb + perms[0]])
        for k in range(1, L):
            acc = acc + plsc.load_gather(hist, [bb + perms[k]])
        hrow[pl.ds(j * L, L)] = acc
    pltpu.sync_copy(hrow, hists_hbm.at[wid])


def _fin_body(hists_ref, hist_ref):
    hist_ref[...] = jnp.sum(hists_ref[...], axis=0, keepdims=True)


_fin = pl.pallas_call(
    _fin_body,
    out_shape=jax.ShapeDtypeStruct((1, NBINS), jnp.float32),
)


def kernel(x):
    mm, mn, mx = _minmax(x.reshape(_MM_MAJ, 8, 128))
    hists = _hist_k(x, mm)
    hist2d = _fin(hists)
    return hist2d.reshape(NBINS), mn.reshape(()), mx.reshape(())


# speedup vs baseline: 1.0669x; 1.0669x over previous
import functools

import jax
import jax.numpy as jnp
from jax import lax
from jax.experimental import pallas as pl
from jax.experimental.pallas import tpu as pltpu
from jax.experimental.pallas import tpu_sc as plsc

NBINS = 2048
NC = 2
NS = 16
L = 16
NW = NC * NS
N = 16777216
PER_W = N // NW
CHUNK = 32768
NCHUNK = PER_W // CHUNK

_MM_MAJ = N // (8 * 128)
_MM_BLK = 1024
_MM_GRID = _MM_MAJ // _MM_BLK


def _mm_body(x_ref, mm_ref, mn_ref, mx_ref, amn_ref, amx_ref):
    i = pl.program_id(0)

    @pl.when(i == 0)
    def _init():
        amn_ref[...] = jnp.full((8, 128), jnp.inf, jnp.float32)
        amx_ref[...] = jnp.full((8, 128), -jnp.inf, jnp.float32)

    v = x_ref[...]
    amn_ref[...] = jnp.minimum(amn_ref[...], jnp.min(v, axis=0))
    amx_ref[...] = jnp.maximum(amx_ref[...], jnp.max(v, axis=0))

    @pl.when(i == _MM_GRID - 1)
    def _fini():
        mn = jnp.min(amn_ref[...])
        mx = jnp.max(amx_ref[...])
        mm_ref[...] = jnp.stack(
            [jnp.full((L,), mn, jnp.float32), jnp.full((L,), mx, jnp.float32)])
        mn_ref[0, 0] = mn
        mx_ref[0, 0] = mx


_minmax = pl.pallas_call(
    _mm_body,
    grid=(_MM_GRID,),
    in_specs=[pl.BlockSpec((_MM_BLK, 8, 128), lambda i: (i, 0, 0))],
    out_specs=(
        pl.BlockSpec((2, L), lambda i: (0, 0)),
        pl.BlockSpec(memory_space=pltpu.SMEM),
        pl.BlockSpec(memory_space=pltpu.SMEM),
    ),
    out_shape=(
        jax.ShapeDtypeStruct((2, L), jnp.float32),
        jax.ShapeDtypeStruct((1, 1), jnp.float32),
        jax.ShapeDtypeStruct((1, 1), jnp.float32),
    ),
    scratch_shapes=[
        pltpu.VMEM((8, 128), jnp.float32),
        pltpu.VMEM((8, 128), jnp.float32),
    ],
)

_mesh = plsc.VectorSubcoreMesh(core_axis_name="c", subcore_axis_name="s")


@functools.partial(
    pl.kernel,
    out_type=jax.ShapeDtypeStruct((NW, NBINS), jnp.float32),
    mesh=_mesh,
    compiler_params=pltpu.CompilerParams(needs_layout_passes=False),
    scratch_types=[
        pltpu.VMEM((2, CHUNK), jnp.float32),
        pltpu.VMEM((L * NBINS,), jnp.float32),
        pltpu.VMEM((NBINS,), jnp.float32),
        pltpu.VMEM((2, L), jnp.float32),
        pltpu.SemaphoreType.DMA,
        pltpu.SemaphoreType.DMA,
    ],
)
def _hist_k(x_hbm, mm_hbm, hists_hbm, buf, hist, hrow, mmv, sem0, sem1):
    wid = lax.axis_index("s") * NC + lax.axis_index("c")
    base = wid * PER_W
    sems = [sem0, sem1]
    cps = [None, None]
    cps[0] = pltpu.async_copy(x_hbm.at[pl.ds(base, CHUNK)], buf.at[0], sem0)

    pltpu.sync_copy(mm_hbm, mmv)
    mn_vec = mmv[0, :]
    mx_vec = mmv[1, :]
    bw = (mx_vec - mn_vec) * jnp.float32(1.0 / NBINS)
    safe_bw = jnp.where(bw <= 0, jnp.float32(1.0), bw)
    inv_vec = jnp.float32(1.0) / safe_bw
    noff_vec = -(mn_vec * inv_vec)

    zero = jnp.zeros((L,), jnp.float32)

    @plsc.parallel_loop(0, (L * NBINS) // L, 1, unroll=8)
    def _zero(j):
        hist[pl.ds(j * L, L)] = zero

    lane = lax.iota(jnp.int32, L)
    ones = jnp.ones((L,), jnp.float32)
    for c in range(NCHUNK):
        b = c % 2
        if c + 1 < NCHUNK:
            cps[1 - b] = pltpu.async_copy(
                x_hbm.at[pl.ds(base + (c + 1) * CHUNK, CHUNK)],
                buf.at[1 - b], sems[1 - b])
        cps[b].wait()

        @plsc.parallel_loop(0, CHUNK // L, 1, unroll=16)
        def _scan(i, b=b):
            v = buf[b, pl.ds(i * L, L)]
            q = jnp.minimum(v * inv_vec + noff_vec, jnp.float32(NBINS - 1))
            plsc.addupdate_scatter(
                hist, [(q.astype(jnp.int32) << 4) | lane], ones)

    perms = [(lane << 4) | ((lane + k) & (L - 1)) for k in range(L)]

    @plsc.parallel_loop(0, NBINS // L, 1, unroll=2)
    def _comb(j):
        bb = j * (L * L)
        acc = plsc.load_gather(hist, [bb + perms[0]])
        for k in range(1, L):
            acc = acc + plsc.load_gather(hist, [bb + perms[k]])
        hrow[pl.ds(j * L, L)] = acc
    pltpu.sync_copy(hrow, hists_hbm.at[wid])


def _fin_body(hists_ref, hist_ref):
    hist_ref[...] = jnp.sum(hists_ref[...], axis=0, keepdims=True)


_fin = pl.pallas_call(
    _fin_body,
    out_shape=jax.ShapeDtypeStruct((1, NBINS), jnp.float32),
)


def kernel(x):
    mm, mn, mx = _minmax(x.reshape(_MM_MAJ, 8, 128))
    hists = _hist_k(x, mm)
    hist2d = _fin(hists)
    return hist2d.reshape(NBINS), mn.reshape(()), mx.reshape(())
